# baseline (speedup 1.0000x reference)
import functools

import jax
import jax.numpy as jnp
from jax import lax
from jax.experimental import pallas as pl
from jax.experimental.pallas import tpu as pltpu
from jax.experimental.pallas import tpu_sc as plsc


D = 128
NC, NS, NLANE = 2, 16, 16


def _make_sc_scatter(E, BM, Bq):
    HD = D // NC
    ROWS = BM // NS
    SCH = 2048
    NSC = E // SCH
    CH = 128
    QCAP = SCH + NLANE
    assert ROWS * NS == BM and NSC * SCH == E and SCH % NLANE == 0
    assert CH % NLANE == 0 and QCAP % NLANE == 0

    mesh = plsc.VectorSubcoreMesh(core_axis_name="c", subcore_axis_name="s")

    @functools.partial(
        pl.kernel,
        out_type=[jax.ShapeDtypeStruct((NC, BM * HD), jnp.float32),
                  jax.ShapeDtypeStruct((BM,), jnp.float32)],
        mesh=mesh,
        compiler_params=pltpu.CompilerParams(needs_layout_passes=False),
        scratch_types=[
            pltpu.VMEM((ROWS * HD,), jnp.float32),
            pltpu.VMEM((ROWS,), jnp.float32),
            pltpu.VMEM((BM,), jnp.float32),
            pltpu.VMEM((Bq * D,), jnp.float32),
            pltpu.VMEM((SCH,), jnp.int32),
            pltpu.VMEM((SCH,), jnp.int32),
            pltpu.VMEM((SCH,), jnp.int32),
            pltpu.VMEM((QCAP,), jnp.int32),
            pltpu.VMEM((QCAP,), jnp.int32),
            pltpu.VMEM((QCAP,), jnp.int32),
            pltpu.VMEM((QCAP,), jnp.int32),
            pltpu.VMEM((CH,), jnp.int32),
            pltpu.VMEM((CH, D), jnp.float32),
            pltpu.SemaphoreType.DMA,
        ])
    def sck(rel_hbm, tail_hbm, head_hbm, bids_hbm, label_hbm, q_hbm,
            nb_hbm, mask_hbm,
            acc_v, mask_v, label_v, q_v, tail_v, head_v, bids_v,
            qe_v, qt_v, qh_v, qb_v, idx_v, rel_v, sem):
        cid = lax.axis_index("c")
        sid = lax.axis_index("s")
        rowbase = sid * ROWS
        dbase = cid * HD

        pltpu.sync_copy(label_hbm, label_v)
        pltpu.sync_copy(q_hbm, q_v)

        zero16 = jnp.zeros((NLANE,), jnp.float32)
        izero16 = jnp.zeros((NLANE,), jnp.int32)
        iota16 = jnp.arange(NLANE, dtype=jnp.int32)

        def zacc(i, c):
            acc_v[pl.ds(i * NLANE, NLANE)] = zero16
            return c
        lax.fori_loop(0, ROWS * HD // NLANE, zacc, 0)

        def zmask(i, c):
            mask_v[pl.ds(i * NLANE, NLANE)] = zero16
            return c
        lax.fori_loop(0, ROWS // NLANE, zmask, 0)

        def zq(i, c):
            qe_v[pl.ds(i * NLANE, NLANE)] = izero16
            qt_v[pl.ds(i * NLANE, NLANE)] = izero16
            qh_v[pl.ds(i * NLANE, NLANE)] = izero16
            qb_v[pl.ds(i * NLANE, NLANE)] = izero16
            return c
        lax.fori_loop(0, QCAP // NLANE, zq, 0)

        def superchunk(si, carry):
            sbase = si * SCH
            pltpu.sync_copy(tail_hbm.at[pl.ds(sbase, SCH)], tail_v)
            pltpu.sync_copy(head_hbm.at[pl.ds(sbase, SCH)], head_v)
            pltpu.sync_copy(bids_hbm.at[pl.ds(sbase, SCH)], bids_v)

            def scan(g, cnt):
                t16 = tail_v[pl.ds(g * NLANE, NLANE)]
                m = (t16 >= rowbase) & (t16 < rowbase + ROWS)
                plsc.store_compressed(qe_v.at[pl.ds(cnt, NLANE)],
                                      iota16 + (sbase + g * NLANE), mask=m)
                plsc.store_compressed(qt_v.at[pl.ds(cnt, NLANE)],
                                      t16 - rowbase, mask=m)
                plsc.store_compressed(qh_v.at[pl.ds(cnt, NLANE)],
                                      head_v[pl.ds(g * NLANE, NLANE)], mask=m)
                plsc.store_compressed(qb_v.at[pl.ds(cnt, NLANE)],
                                      bids_v[pl.ds(g * NLANE, NLANE)], mask=m)
                return cnt + plsc.all_reduce_population_count(m)[0]
            cnt = lax.fori_loop(0, SCH // NLANE, scan, jnp.int32(0))

            nq = (cnt + (CH - 1)) // CH

            def chunk(k, cc):
                base = k * CH
                for g in range(CH // NLANE):
                    idx_v[pl.ds(g * NLANE, NLANE)] = (
                        qe_v[pl.ds(base + g * NLANE, NLANE)])
                pltpu.async_copy(rel_hbm.at[idx_v], rel_v, sem).wait()

                for g in range(CH // NLANE):
                    lanebase = base + g * NLANE
                    h16 = qh_v[pl.ds(lanebase, NLANE)]
                    t16 = qt_v[pl.ds(lanebase, NLANE)]
                    b16 = qb_v[pl.ds(lanebase, NLANE)]
                    p16 = plsc.load_gather(label_v, [h16])
                    p16 = jnp.where((lanebase + iota16) < cnt, p16, 0.0)

                    @pl.when(cid == 0)
                    def _mask_adds(t16=t16, p16=p16):
                        for j in range(NLANE):
                            plsc.addupdate_scatter(mask_v, [t16], p16,
                                                   mask=(iota16 == j))

                    for j in range(NLANE):
                        p = p16[j]
                        b = b16[j]
                        tl = t16[j]
                        row = g * NLANE + j
                        for c in range(HD // NLANE):
                            relv = rel_v[row, pl.ds(dbase + c * NLANE, NLANE)]
                            qv = plsc.load_gather(
                                q_v, [iota16 + (b * D + dbase + c * NLANE)])
                            plsc.addupdate_scatter(
                                acc_v, [iota16 + (tl * HD + c * NLANE)],
                                p * jnp.maximum(qv * relv, 0.0))
                return cc
            lax.fori_loop(0, nq, chunk, 0)
            return carry
        lax.fori_loop(0, NSC, superchunk, 0)

        pltpu.sync_copy(acc_v, nb_hbm.at[cid, pl.ds(rowbase * HD, ROWS * HD)])

        @pl.when(cid == 0)
        def _mask_out():
            pltpu.sync_copy(mask_v, mask_hbm.at[pl.ds(rowbase, ROWS)])

    return sck


def _make_sc_kge_gather(E, BM):
    EPT = E // (NC * NS)
    SCH = 2000
    CH = 80
    NSC = EPT // SCH
    NCH = SCH // CH
    assert EPT * NC * NS == E and NSC * SCH == EPT
    assert SCH % CH == 0

    mesh = plsc.VectorSubcoreMesh(core_axis_name="c", subcore_axis_name="s")

    @functools.partial(
        pl.kernel,
        out_type=jax.ShapeDtypeStruct((E, D), jnp.float32),
        mesh=mesh,
        compiler_params=pltpu.CompilerParams(needs_layout_passes=False),
        scratch_types=[
            pltpu.VMEM((SCH,), jnp.int32),
            pltpu.VMEM((SCH,), jnp.int32),
            pltpu.VMEM((CH,), jnp.int32),
            pltpu.VMEM((CH,), jnp.int32),
            pltpu.VMEM((CH, D), jnp.float32),
            pltpu.VMEM((CH, D), jnp.float32),
            pltpu.SemaphoreType.DMA,
        ])
    def kge(ef_hbm, head_hbm, tail_hbm, u_hbm,
            head_v, tail_v, idxh_v, idxt_v, hbuf, tbuf, sem):
        cid = lax.axis_index("c")
        sid = lax.axis_index("s")
        wid = sid * NC + cid
        ebase = wid * EPT

        def superchunk(si, carry):
            sbase = ebase + si * SCH
            pltpu.sync_copy(head_hbm.at[pl.ds(sbase, SCH)], head_v)
            pltpu.sync_copy(tail_hbm.at[pl.ds(sbase, SCH)], tail_v)

            def chunk(ci, ccarry):
                loff = ci * CH
                for g in range(CH // NLANE):
                    idxh_v[pl.ds(g * NLANE, NLANE)] = (
                        head_v[pl.ds(loff + g * NLANE, NLANE)])
                    idxt_v[pl.ds(g * NLANE, NLANE)] = (
                        tail_v[pl.ds(loff + g * NLANE, NLANE)])
                pltpu.async_copy(ef_hbm.at[idxh_v], hbuf, sem).wait()
                pltpu.async_copy(ef_hbm.at[idxt_v], tbuf, sem).wait()
                for r in range(CH):
                    for c in range(D // NLANE):
                        sl = pl.ds(c * NLANE, NLANE)
                        hbuf[r, sl] = hbuf[r, sl] * tbuf[r, sl]
                pltpu.sync_copy(hbuf, u_hbm.at[pl.ds(sbase + loff, CH)])
                return ccarry
            lax.fori_loop(0, NCH, chunk, 0)
            return carry
        lax.fori_loop(0, NSC, superchunk, 0)

    return kge


def _ln(x, g, b, eps=1e-5):
    mu = jnp.mean(x, axis=-1, keepdims=True)
    var = jnp.mean((x - mu) ** 2, axis=-1, keepdims=True)
    return (x - mu) * jax.lax.rsqrt(var + eps) * g + b


def _ln_body(x_ref, g_ref, b_ref, o_ref):
    o_ref[...] = _ln(x_ref[...], g_ref[...], b_ref[...])


def _ln_rows(x, g, b, block=1000):
    n = x.shape[0]
    grid = n // block
    return pl.pallas_call(
        _ln_body,
        grid=(grid,),
        in_specs=[
            pl.BlockSpec((block, D), lambda i: (i, 0)),
            pl.BlockSpec((1, D), lambda i: (0, 0)),
            pl.BlockSpec((1, D), lambda i: (0, 0)),
        ],
        out_specs=pl.BlockSpec((block, D), lambda i: (i, 0)),
        out_shape=jax.ShapeDtypeStruct((n, D), jnp.float32),
    )(x, g.reshape(1, D), b.reshape(1, D))


def _kge_body(u_ref, rel_ref, w_ref, b_ref, acc_ref):
    i = pl.program_id(0)

    @pl.when(i == 0)
    def _init():
        acc_ref[...] = jnp.zeros_like(acc_ref)

    u = u_ref[...]
    rel = rel_ref[...]
    v = jax.lax.dot_general(u, w_ref[...], (((1,), (0,)), ((), ())),
                            preferred_element_type=jnp.float32)
    s = jnp.sum(v * rel, axis=1) + jnp.sum(u * b_ref[...], axis=1)
    nll = -jnp.log(jax.nn.sigmoid(s) + 1e-20)
    acc_ref[...] += jnp.sum(nll).reshape(1, 1)


def _kge_loss_sum(u, rel, w_rel, b_rel, block=2000):
    e = u.shape[0]
    grid = e // block
    out = pl.pallas_call(
        _kge_body,
        grid=(grid,),
        in_specs=[
            pl.BlockSpec((block, D), lambda i: (i, 0)),
            pl.BlockSpec((block, D), lambda i: (i, 0)),
            pl.BlockSpec((D, D), lambda i: (0, 0)),
            pl.BlockSpec((1, D), lambda i: (0, 0)),
        ],
        out_specs=pl.BlockSpec((1, 1), lambda i: (0, 0)),
        out_shape=jax.ShapeDtypeStruct((1, 1), jnp.float32),
    )(u, rel, w_rel, b_rel.reshape(1, D))
    return out[0, 0]


def _step_body(nb_ref, g_ref, b_ref,
               wih0_ref, whh0_ref, bhh0_ref,
               wih1_ref, whh1_ref, bhh1_ref,
               ws_ref, bs_ref, h1_ref, h2_ref,
               h1o_ref, h2o_ref, sco_ref):
    g = g_ref[...]
    b = b_ref[...]
    nb = jnp.concatenate([nb_ref[0], nb_ref[1]], axis=-1)
    x = _ln(nb, g, b)

    def gru_layer(x_in, h_prev, wih, whh, bhh):
        xg = jax.lax.dot_general(x_in, wih, (((1,), (1,)), ((), ())),
                                 preferred_element_type=jnp.float32)
        hg = jax.lax.dot_general(h_prev, whh, (((1,), (1,)), ((), ())),
                                 preferred_element_type=jnp.float32) + bhh
        upd = jax.nn.sigmoid(xg[:, 0:D] + hg[:, 0:D])
        rst = jax.nn.sigmoid(xg[:, D:2 * D] + hg[:, D:2 * D])
        mem = jnp.tanh(xg[:, 2 * D:3 * D] + rst * hg[:, 2 * D:3 * D])
        return _ln((1.0 - upd) * mem + upd * h_prev, g, b)

    h1 = gru_layer(x, h1_ref[...], wih0_ref[...], whh0_ref[...], bhh0_ref[...])
    h2 = gru_layer(h1, h2_ref[...], wih1_ref[...], whh1_ref[...], bhh1_ref[...])
    h1o_ref[...] = h1
    h2o_ref[...] = h2
    sc = jnp.sum(h2 * ws_ref[...], axis=1)[None, :] + bs_ref[...]
    sco_ref[...] = sc[None, :, :]


def _step_dense(nb_parts, ln_g, ln_b, wih0, whh0, bhh0, wih1, whh1, bhh1,
                w_score, b_score, h1_prev, h2_prev, block=1000):
    p, n, hd = nb_parts.shape
    grid = n // block
    h1o, h2o, sco = pl.pallas_call(
        _step_body,
        grid=(grid,),
        in_specs=[
            pl.BlockSpec((p, block, hd), lambda i: (0, i, 0)),
            pl.BlockSpec((1, D), lambda i: (0, 0)),
            pl.BlockSpec((1, D), lambda i: (0, 0)),
            pl.BlockSpec((3 * D, D), lambda i: (0, 0)),
            pl.BlockSpec((3 * D, D), lambda i: (0, 0)),
            pl.BlockSpec((1, 3 * D), lambda i: (0, 0)),
            pl.BlockSpec((3 * D, D), lambda i: (0, 0)),
            pl.BlockSpec((3 * D, D), lambda i: (0, 0)),
            pl.BlockSpec((1, 3 * D), lambda i: (0, 0)),
            pl.BlockSpec((1, D), lambda i: (0, 0)),
            pl.BlockSpec((1, 1), lambda i: (0, 0)),
            pl.BlockSpec((block, D), lambda i: (i, 0)),
            pl.BlockSpec((block, D), lambda i: (i, 0)),
        ],
        out_specs=[
            pl.BlockSpec((block, D), lambda i: (i, 0)),
            pl.BlockSpec((block, D), lambda i: (i, 0)),
            pl.BlockSpec((1, 1, block), lambda i: (i, 0, 0)),
        ],
        out_shape=[
            jax.ShapeDtypeStruct((n, D), jnp.float32),
            jax.ShapeDtypeStruct((n, D), jnp.float32),
            jax.ShapeDtypeStruct((grid, 1, block), jnp.float32),
        ],
    )(nb_parts, ln_g.reshape(1, D), ln_b.reshape(1, D),
      wih0, whh0, bhh0.reshape(1, 3 * D), wih1, whh1, bhh1.reshape(1, 3 * D),
      w_score, b_score.reshape(1, 1), h1_prev, h2_prev)
    return h1o, h2o, sco.reshape(n)


def _mask_softmax_body(sc_ref, mk_ref, lbl_ref, em_ref, out_ref):
    im = ((mk_ref[...] + lbl_ref[...]) > 1e-8).astype(jnp.float32) * em_ref[...]
    s = im * sc_ref[...] + (1.0 - im) * (-1e20)
    m = jnp.max(s, axis=1, keepdims=True)
    ex = jnp.exp(s - m)
    out_ref[...] = ex / jnp.sum(ex, axis=1, keepdims=True)


def _mask_softmax(score_bm, mask_bm, prev_label, entity_mask):
    bq, mq = score_bm.shape
    return pl.pallas_call(
        _mask_softmax_body,
        in_specs=[
            pl.BlockSpec((bq, mq), lambda: (0, 0)),
            pl.BlockSpec((bq, mq), lambda: (0, 0)),
            pl.BlockSpec((bq, mq), lambda: (0, 0)),
            pl.BlockSpec((bq, mq), lambda: (0, 0)),
        ],
        out_specs=pl.BlockSpec((bq, mq), lambda: (0, 0)),
        out_shape=jax.ShapeDtypeStruct((bq, mq), jnp.float32),
    )(score_bm, mask_bm, prev_label, entity_mask)


def _ffn_body(h_ref, w_ref, b_ref, o_ref):
    o_ref[...] = jax.lax.dot_general(
        h_ref[...], w_ref[...], (((1,), (1,)), ((), ())),
        preferred_element_type=jnp.float32) + b_ref[...]


def _ffn(h, w, b, block=1000):
    n = h.shape[0]
    grid = n // block
    return pl.pallas_call(
        _ffn_body,
        grid=(grid,),
        in_specs=[
            pl.BlockSpec((block, D), lambda i: (i, 0)),
            pl.BlockSpec((D, D), lambda i: (0, 0)),
            pl.BlockSpec((1, D), lambda i: (0, 0)),
        ],
        out_specs=pl.BlockSpec((block, D), lambda i: (i, 0)),
        out_shape=jax.ShapeDtypeStruct((n, D), jnp.float32),
    )(h, w, b.reshape(1, D))


def kernel(instructions, entity_emb, fact_relations, topic_label, entity_mask,
           batch_ids, head2edge, tail2edge, ln_g, ln_b, W_rel, b_rel,
           W_ih_0, W_hh_0, b_hh_0, W_ih_1, W_hh_1, b_hh_1,
           W_score, b_score, W_ffn, b_ffn):
    S = instructions.shape[0]
    Bq, Mq, Dq = entity_emb.shape
    BM = Bq * Mq
    E = fact_relations.shape[0]

    ef = _ln_rows(entity_emb.reshape(BM, Dq), ln_g, ln_b)

    u = _make_sc_kge_gather(E, BM)(ef, head2edge, tail2edge)
    kge_loss = 2.0 * _kge_loss_sum(u, fact_relations, W_rel, b_rel) / E

    h1 = ef
    h2 = ef
    ent_label = topic_label
    labels = []
    sc_scatter = _make_sc_scatter(E, BM, Bq)
    for i in range(S):
        q = instructions[i]
        nb_flat, mask_flat = sc_scatter(
            fact_relations, tail2edge, head2edge, batch_ids,
            ent_label.reshape(BM), q.reshape(Bq * Dq))
        nb_parts = nb_flat.reshape(NC, BM, D // NC)
        h1, h2, score = _step_dense(
            nb_parts, ln_g, ln_b, W_ih_0, W_hh_0, b_hh_0,
            W_ih_1, W_hh_1, b_hh_1, W_score, b_score, h1, h2)
        ent_label = _mask_softmax(score.reshape(Bq, Mq),
                                  mask_flat.reshape(Bq, Mq),
                                  ent_label, entity_mask)
        labels.append(ent_label)

    final = _ffn(h2, W_ffn, b_ffn).reshape(Bq, Mq, Dq)
    return (jnp.stack(labels, axis=0), final, jnp.stack([kge_loss]))


# speedup vs baseline: 1.0845x; 1.0845x over previous
import functools

import jax
import jax.numpy as jnp
from jax import lax
from jax.experimental import pallas as pl
from jax.experimental.pallas import tpu as pltpu
from jax.experimental.pallas import tpu_sc as plsc


D = 128
NC, NS, NLANE = 2, 16, 16


def _make_sc_scatter(E, BM, Bq):
    NT = NC * NS
    ROWS = BM // NT
    MB = 512
    SCH = 2048
    NSC = E // SCH
    CH = 128
    QCAP = SCH + NLANE
    assert ROWS * NT == BM and NSC * SCH == E and SCH % NLANE == 0
    assert CH % NLANE == 0 and QCAP % NLANE == 0 and ROWS <= MB

    mesh = plsc.VectorSubcoreMesh(core_axis_name="c", subcore_axis_name="s")

    @functools.partial(
        pl.kernel,
        out_type=[jax.ShapeDtypeStruct((BM * D,), jnp.float32),
                  jax.ShapeDtypeStruct((NT, MB), jnp.float32)],
        mesh=mesh,
        compiler_params=pltpu.CompilerParams(needs_layout_passes=False),
        scratch_types=[
            pltpu.VMEM((ROWS * D,), jnp.float32),
            pltpu.VMEM((MB,), jnp.float32),
            pltpu.VMEM((BM,), jnp.float32),
            pltpu.VMEM((Bq * D,), jnp.float32),
            pltpu.VMEM((SCH,), jnp.int32),
            pltpu.VMEM((SCH,), jnp.int32),
            pltpu.VMEM((SCH,), jnp.int32),
            pltpu.VMEM((QCAP,), jnp.int32),
            pltpu.VMEM((QCAP,), jnp.int32),
            pltpu.VMEM((QCAP,), jnp.int32),
            pltpu.VMEM((QCAP,), jnp.int32),
            pltpu.VMEM((CH,), jnp.int32),
            pltpu.VMEM((CH, D), jnp.float32),
            pltpu.SemaphoreType.DMA,
        ])
    def sck(rel_hbm, tail_hbm, head_hbm, bids_hbm, label_hbm, q_hbm,
            nb_hbm, mask_hbm,
            acc_v, mask_v, label_v, q_v, tail_v, head_v, bids_v,
            qe_v, qt_v, qh_v, qb_v, idx_v, rel_v, sem):
        cid = lax.axis_index("c")
        sid = lax.axis_index("s")
        rowbase = (cid * NS + sid) * ROWS

        pltpu.sync_copy(label_hbm, label_v)
        pltpu.sync_copy(q_hbm, q_v)

        zero16 = jnp.zeros((NLANE,), jnp.float32)
        izero16 = jnp.zeros((NLANE,), jnp.int32)
        iota16 = jnp.arange(NLANE, dtype=jnp.int32)

        def zacc(i, c):
            acc_v[pl.ds(i * NLANE, NLANE)] = zero16
            return c
        lax.fori_loop(0, ROWS * D // NLANE, zacc, 0)

        def zmask(i, c):
            mask_v[pl.ds(i * NLANE, NLANE)] = zero16
            return c
        lax.fori_loop(0, MB // NLANE, zmask, 0)

        def zq(i, c):
            qe_v[pl.ds(i * NLANE, NLANE)] = izero16
            qt_v[pl.ds(i * NLANE, NLANE)] = izero16
            qh_v[pl.ds(i * NLANE, NLANE)] = izero16
            qb_v[pl.ds(i * NLANE, NLANE)] = izero16
            return c
        lax.fori_loop(0, QCAP // NLANE, zq, 0)

        def superchunk(si, carry):
            sbase = si * SCH
            pltpu.sync_copy(tail_hbm.at[pl.ds(sbase, SCH)], tail_v)
            pltpu.sync_copy(head_hbm.at[pl.ds(sbase, SCH)], head_v)
            pltpu.sync_copy(bids_hbm.at[pl.ds(sbase, SCH)], bids_v)

            def scan(g, cnt):
                t16 = tail_v[pl.ds(g * NLANE, NLANE)]
                m = (t16 >= rowbase) & (t16 < rowbase + ROWS)
                plsc.store_compressed(qe_v.at[pl.ds(cnt, NLANE)],
                                      iota16 + (sbase + g * NLANE), mask=m)
                plsc.store_compressed(qt_v.at[pl.ds(cnt, NLANE)],
                                      t16 - rowbase, mask=m)
                plsc.store_compressed(qh_v.at[pl.ds(cnt, NLANE)],
                                      head_v[pl.ds(g * NLANE, NLANE)], mask=m)
                plsc.store_compressed(qb_v.at[pl.ds(cnt, NLANE)],
                                      bids_v[pl.ds(g * NLANE, NLANE)], mask=m)
                return cnt + plsc.all_reduce_population_count(m)[0]
            cnt = lax.fori_loop(0, SCH // NLANE, scan, jnp.int32(0))

            nq = (cnt + (CH - 1)) // CH

            def chunk(k, cc):
                base = k * CH
                for g in range(CH // NLANE):
                    idx_v[pl.ds(g * NLANE, NLANE)] = (
                        qe_v[pl.ds(base + g * NLANE, NLANE)])
                pltpu.async_copy(rel_hbm.at[idx_v], rel_v, sem).wait()

                for g in range(CH // NLANE):
                    lanebase = base + g * NLANE
                    h16 = qh_v[pl.ds(lanebase, NLANE)]
                    t16 = qt_v[pl.ds(lanebase, NLANE)]
                    b16 = qb_v[pl.ds(lanebase, NLANE)]
                    p16 = plsc.load_gather(label_v, [h16])
                    p16 = jnp.where((lanebase + iota16) < cnt, p16, 0.0)

                    for j in range(NLANE):
                        plsc.addupdate_scatter(mask_v, [t16], p16,
                                               mask=(iota16 == j))

                    for j in range(NLANE):
                        p = p16[j]
                        b = b16[j]
                        tl = t16[j]
                        row = g * NLANE + j
                        for c in range(D // NLANE):
                            relv = rel_v[row, pl.ds(c * NLANE, NLANE)]
                            qv = q_v[pl.ds(b * D + c * NLANE, NLANE)]
                            plsc.addupdate_scatter(
                                acc_v, [iota16 + (tl * D + c * NLANE)],
                                p * jnp.maximum(qv * relv, 0.0))
                return cc
            lax.fori_loop(0, nq, chunk, 0)
            return carry
        lax.fori_loop(0, NSC, superchunk, 0)

        pltpu.sync_copy(acc_v, nb_hbm.at[pl.ds(rowbase * D, ROWS * D)])
        pltpu.sync_copy(mask_v, mask_hbm.at[cid * NS + sid])

    return sck


def _make_sc_kge_gather(E, BM):
    EPT = E // (NC * NS)
    SCH = 2000
    CH = 80
    NSC = EPT // SCH
    NCH = SCH // CH
    assert EPT * NC * NS == E and NSC * SCH == EPT
    assert SCH % CH == 0

    mesh = plsc.VectorSubcoreMesh(core_axis_name="c", subcore_axis_name="s")

    @functools.partial(
        pl.kernel,
        out_type=jax.ShapeDtypeStruct((E, D), jnp.float32),
        mesh=mesh,
        compiler_params=pltpu.CompilerParams(needs_layout_passes=False),
        scratch_types=[
            pltpu.VMEM((SCH,), jnp.int32),
            pltpu.VMEM((SCH,), jnp.int32),
            pltpu.VMEM((CH,), jnp.int32),
            pltpu.VMEM((CH,), jnp.int32),
            pltpu.VMEM((CH, D), jnp.float32),
            pltpu.VMEM((CH, D), jnp.float32),
            pltpu.SemaphoreType.DMA,
        ])
    def kge(ef_hbm, head_hbm, tail_hbm, u_hbm,
            head_v, tail_v, idxh_v, idxt_v, hbuf, tbuf, sem):
        cid = lax.axis_index("c")
        sid = lax.axis_index("s")
        wid = sid * NC + cid
        ebase = wid * EPT

        def superchunk(si, carry):
            sbase = ebase + si * SCH
            pltpu.sync_copy(head_hbm.at[pl.ds(sbase, SCH)], head_v)
            pltpu.sync_copy(tail_hbm.at[pl.ds(sbase, SCH)], tail_v)

            def chunk(ci, ccarry):
                loff = ci * CH
                for g in range(CH // NLANE):
                    idxh_v[pl.ds(g * NLANE, NLANE)] = (
                        head_v[pl.ds(loff + g * NLANE, NLANE)])
                    idxt_v[pl.ds(g * NLANE, NLANE)] = (
                        tail_v[pl.ds(loff + g * NLANE, NLANE)])
                pltpu.async_copy(ef_hbm.at[idxh_v], hbuf, sem).wait()
                pltpu.async_copy(ef_hbm.at[idxt_v], tbuf, sem).wait()
                for r in range(CH):
                    for c in range(D // NLANE):
                        sl = pl.ds(c * NLANE, NLANE)
                        hbuf[r, sl] = hbuf[r, sl] * tbuf[r, sl]
                pltpu.sync_copy(hbuf, u_hbm.at[pl.ds(sbase + loff, CH)])
                return ccarry
            lax.fori_loop(0, NCH, chunk, 0)
            return carry
        lax.fori_loop(0, NSC, superchunk, 0)

    return kge


def _ln(x, g, b, eps=1e-5):
    mu = jnp.mean(x, axis=-1, keepdims=True)
    var = jnp.mean((x - mu) ** 2, axis=-1, keepdims=True)
    return (x - mu) * jax.lax.rsqrt(var + eps) * g + b


def _ln_body(x_ref, g_ref, b_ref, o_ref):
    o_ref[...] = _ln(x_ref[...], g_ref[...], b_ref[...])


def _ln_rows(x, g, b, block=1000):
    n = x.shape[0]
    grid = n // block
    return pl.pallas_call(
        _ln_body,
        grid=(grid,),
        in_specs=[
            pl.BlockSpec((block, D), lambda i: (i, 0)),
            pl.BlockSpec((1, D), lambda i: (0, 0)),
            pl.BlockSpec((1, D), lambda i: (0, 0)),
        ],
        out_specs=pl.BlockSpec((block, D), lambda i: (i, 0)),
        out_shape=jax.ShapeDtypeStruct((n, D), jnp.float32),
    )(x, g.reshape(1, D), b.reshape(1, D))


def _kge_body(u_ref, rel_ref, w_ref, b_ref, acc_ref):
    i = pl.program_id(0)

    @pl.when(i == 0)
    def _init():
        acc_ref[...] = jnp.zeros_like(acc_ref)

    u = u_ref[...]
    rel = rel_ref[...]
    v = jax.lax.dot_general(u, w_ref[...], (((1,), (0,)), ((), ())),
                            preferred_element_type=jnp.float32)
    s = jnp.sum(v * rel, axis=1) + jnp.sum(u * b_ref[...], axis=1)
    nll = -jnp.log(jax.nn.sigmoid(s) + 1e-20)
    acc_ref[...] += jnp.sum(nll).reshape(1, 1)


def _kge_loss_sum(u, rel, w_rel, b_rel, block=2000):
    e = u.shape[0]
    grid = e // block
    out = pl.pallas_call(
        _kge_body,
        grid=(grid,),
        in_specs=[
            pl.BlockSpec((block, D), lambda i: (i, 0)),
            pl.BlockSpec((block, D), lambda i: (i, 0)),
            pl.BlockSpec((D, D), lambda i: (0, 0)),
            pl.BlockSpec((1, D), lambda i: (0, 0)),
        ],
        out_specs=pl.BlockSpec((1, 1), lambda i: (0, 0)),
        out_shape=jax.ShapeDtypeStruct((1, 1), jnp.float32),
    )(u, rel, w_rel, b_rel.reshape(1, D))
    return out[0, 0]


def _step_body(nb_ref, g_ref, b_ref,
               wih0_ref, whh0_ref, bhh0_ref,
               wih1_ref, whh1_ref, bhh1_ref,
               ws_ref, bs_ref, h1_ref, h2_ref,
               h1o_ref, h2o_ref, sco_ref):
    g = g_ref[...]
    b = b_ref[...]
    x = _ln(nb_ref[...], g, b)

    def gru_layer(x_in, h_prev, wih, whh, bhh):
        xg = jax.lax.dot_general(x_in, wih, (((1,), (1,)), ((), ())),
                                 preferred_element_type=jnp.float32)
        hg = jax.lax.dot_general(h_prev, whh, (((1,), (1,)), ((), ())),
                                 preferred_element_type=jnp.float32) + bhh
        upd = jax.nn.sigmoid(xg[:, 0:D] + hg[:, 0:D])
        rst = jax.nn.sigmoid(xg[:, D:2 * D] + hg[:, D:2 * D])
        mem = jnp.tanh(xg[:, 2 * D:3 * D] + rst * hg[:, 2 * D:3 * D])
        return _ln((1.0 - upd) * mem + upd * h_prev, g, b)

    h1 = gru_layer(x, h1_ref[...], wih0_ref[...], whh0_ref[...], bhh0_ref[...])
    h2 = gru_layer(h1, h2_ref[...], wih1_ref[...], whh1_ref[...], bhh1_ref[...])
    h1o_ref[...] = h1
    h2o_ref[...] = h2
    sc = jnp.sum(h2 * ws_ref[...], axis=1)[None, :] + bs_ref[...]
    sco_ref[...] = sc[None, :, :]


def _step_dense(nb, ln_g, ln_b, wih0, whh0, bhh0, wih1, whh1, bhh1,
                w_score, b_score, h1_prev, h2_prev, block=1000):
    n = nb.shape[0]
    grid = n // block
    h1o, h2o, sco = pl.pallas_call(
        _step_body,
        grid=(grid,),
        in_specs=[
            pl.BlockSpec((block, D), lambda i: (i, 0)),
            pl.BlockSpec((1, D), lambda i: (0, 0)),
            pl.BlockSpec((1, D), lambda i: (0, 0)),
            pl.BlockSpec((3 * D, D), lambda i: (0, 0)),
            pl.BlockSpec((3 * D, D), lambda i: (0, 0)),
            pl.BlockSpec((1, 3 * D), lambda i: (0, 0)),
            pl.BlockSpec((3 * D, D), lambda i: (0, 0)),
            pl.BlockSpec((3 * D, D), lambda i: (0, 0)),
            pl.BlockSpec((1, 3 * D), lambda i: (0, 0)),
            pl.BlockSpec((1, D), lambda i: (0, 0)),
            pl.BlockSpec((1, 1), lambda i: (0, 0)),
            pl.BlockSpec((block, D), lambda i: (i, 0)),
            pl.BlockSpec((block, D), lambda i: (i, 0)),
        ],
        out_specs=[
            pl.BlockSpec((block, D), lambda i: (i, 0)),
            pl.BlockSpec((block, D), lambda i: (i, 0)),
            pl.BlockSpec((1, 1, block), lambda i: (i, 0, 0)),
        ],
        out_shape=[
            jax.ShapeDtypeStruct((n, D), jnp.float32),
            jax.ShapeDtypeStruct((n, D), jnp.float32),
            jax.ShapeDtypeStruct((grid, 1, block), jnp.float32),
        ],
    )(nb, ln_g.reshape(1, D), ln_b.reshape(1, D),
      wih0, whh0, bhh0.reshape(1, 3 * D), wih1, whh1, bhh1.reshape(1, 3 * D),
      w_score, b_score.reshape(1, 1), h1_prev, h2_prev)
    return h1o, h2o, sco.reshape(n)


def _mask_softmax_body(sc_ref, mk_ref, lbl_ref, em_ref, out_ref):
    im = ((mk_ref[...] + lbl_ref[...]) > 1e-8).astype(jnp.float32) * em_ref[...]
    s = im * sc_ref[...] + (1.0 - im) * (-1e20)
    m = jnp.max(s, axis=1, keepdims=True)
    ex = jnp.exp(s - m)
    out_ref[...] = ex / jnp.sum(ex, axis=1, keepdims=True)


def _mask_softmax(score_bm, mask_bm, prev_label, entity_mask):
    bq, mq = score_bm.shape
    return pl.pallas_call(
        _mask_softmax_body,
        in_specs=[
            pl.BlockSpec((bq, mq), lambda: (0, 0)),
            pl.BlockSpec((bq, mq), lambda: (0, 0)),
            pl.BlockSpec((bq, mq), lambda: (0, 0)),
            pl.BlockSpec((bq, mq), lambda: (0, 0)),
        ],
        out_specs=pl.BlockSpec((bq, mq), lambda: (0, 0)),
        out_shape=jax.ShapeDtypeStruct((bq, mq), jnp.float32),
    )(score_bm, mask_bm, prev_label, entity_mask)


def _ffn_body(h_ref, w_ref, b_ref, o_ref):
    o_ref[...] = jax.lax.dot_general(
        h_ref[...], w_ref[...], (((1,), (1,)), ((), ())),
        preferred_element_type=jnp.float32) + b_ref[...]


def _ffn(h, w, b, block=1000):
    n = h.shape[0]
    grid = n // block
    return pl.pallas_call(
        _ffn_body,
        grid=(grid,),
        in_specs=[
            pl.BlockSpec((block, D), lambda i: (i, 0)),
            pl.BlockSpec((D, D), lambda i: (0, 0)),
            pl.BlockSpec((1, D), lambda i: (0, 0)),
        ],
        out_specs=pl.BlockSpec((block, D), lambda i: (i, 0)),
        out_shape=jax.ShapeDtypeStruct((n, D), jnp.float32),
    )(h, w, b.reshape(1, D))


def kernel(instructions, entity_emb, fact_relations, topic_label, entity_mask,
           batch_ids, head2edge, tail2edge, ln_g, ln_b, W_rel, b_rel,
           W_ih_0, W_hh_0, b_hh_0, W_ih_1, W_hh_1, b_hh_1,
           W_score, b_score, W_ffn, b_ffn):
    S = instructions.shape[0]
    Bq, Mq, Dq = entity_emb.shape
    BM = Bq * Mq
    E = fact_relations.shape[0]

    ef = _ln_rows(entity_emb.reshape(BM, Dq), ln_g, ln_b)

    u = _make_sc_kge_gather(E, BM)(ef, head2edge, tail2edge)
    kge_loss = 2.0 * _kge_loss_sum(u, fact_relations, W_rel, b_rel) / E

    h1 = ef
    h2 = ef
    ent_label = topic_label
    labels = []
    sc_scatter = _make_sc_scatter(E, BM, Bq)
    for i in range(S):
        q = instructions[i]
        nb_flat, mask_flat = sc_scatter(
            fact_relations, tail2edge, head2edge, batch_ids,
            ent_label.reshape(BM), q.reshape(Bq * Dq))
        h1, h2, score = _step_dense(
            nb_flat.reshape(BM, D), ln_g, ln_b, W_ih_0, W_hh_0, b_hh_0,
            W_ih_1, W_hh_1, b_hh_1, W_score, b_score, h1, h2)
        mask_bm = mask_flat[:, :BM // mask_flat.shape[0]].reshape(Bq, Mq)
        ent_label = _mask_softmax(score.reshape(Bq, Mq), mask_bm,
                                  ent_label, entity_mask)
        labels.append(ent_label)

    final = _ffn(h2, W_ffn, b_ffn).reshape(Bq, Mq, Dq)
    return (jnp.stack(labels, axis=0), final, jnp.stack([kge_loss]))
